# baseline (speedup 1.0000x reference)
import functools

import jax
import jax.numpy as jnp
from jax import lax
from jax.experimental import pallas as pl
from jax.experimental.pallas import tpu as pltpu
from jax.experimental.pallas import tpu_sc as plsc

B = 16384
D = 64

_GATHER_DNUMS = lax.GatherDimensionNumbers(
    offset_dims=(), collapsed_slice_dims=(0,), start_index_map=(0,))


def _shuffle(x, idx):
    return lax.gather(
        x, idx[:, None], _GATHER_DNUMS, slice_sizes=(1,),
        mode=lax.GatherScatterMode.PROMISE_IN_BOUNDS)

_info = plsc.get_sparse_core_info()
NC, NS, L = _info.num_cores, _info.num_subcores, _info.num_lanes
NW = NC * NS
BPW = B // NW
CHUNK = 128
NCHUNK = BPW // CHUNK
NG = BPW // 16
NCOL = D // L


def _distmult_body(e_idc, p_idc, u_idc, node_tab, edge_tab, out_hbm,
                   eidx, pidx, uidx, e_v, p_v, u_v, out_v, sem):
    wid = lax.axis_index("s") * NC + lax.axis_index("c")
    base = wid * BPW

    pltpu.sync_copy(e_idc.at[pl.ds(base, BPW)], eidx)
    pltpu.sync_copy(p_idc.at[pl.ds(base, BPW)], pidx)
    pltpu.sync_copy(u_idc.at[pl.ds(base, BPW)], uidx)

    copies = []
    for k in range(NCHUNK):
        s = pl.ds(k * CHUNK, CHUNK)
        copies.append(pltpu.async_copy(node_tab.at[eidx.at[s]], e_v.at[s], sem))
        copies.append(pltpu.async_copy(edge_tab.at[pidx.at[s]], p_v.at[s], sem))
        copies.append(pltpu.async_copy(node_tab.at[uidx.at[s]], u_v.at[s], sem))
    for c in copies:
        c.wait()

    lane = lax.iota(jnp.int32, L)

    def group(g, carry):
        tot = jnp.zeros((L,), jnp.float32)
        for r in range(16):
            row = g * 16 + r
            acc = (e_v[row, pl.ds(0, L)] * p_v[row, pl.ds(0, L)]
                   * u_v[row, pl.ds(0, L)])
            for c in range(1, NCOL):
                acc = acc + (e_v[row, pl.ds(c * L, L)]
                             * p_v[row, pl.ds(c * L, L)]
                             * u_v[row, pl.ds(c * L, L)])
            for sh in (8, 4, 2, 1):
                acc = acc + _shuffle(acc, lane ^ sh)
            tot = jnp.where(lane == r, acc, tot)
        out_v[pl.ds(g * 16, 16)] = tot
        return carry

    lax.fori_loop(0, NG, group, 0)

    pltpu.sync_copy(out_v, out_hbm.at[pl.ds(base, BPW)])


_distmult = pl.kernel(
    _distmult_body,
    out_type=jax.ShapeDtypeStruct((B,), jnp.float32),
    mesh=plsc.VectorSubcoreMesh(core_axis_name="c", subcore_axis_name="s"),
    compiler_params=pltpu.CompilerParams(use_tc_tiling_on_sc=False),
    scratch_types=[
        pltpu.VMEM((BPW,), jnp.int32),
        pltpu.VMEM((BPW,), jnp.int32),
        pltpu.VMEM((BPW,), jnp.int32),
        pltpu.VMEM((BPW, D), jnp.float32),
        pltpu.VMEM((BPW, D), jnp.float32),
        pltpu.VMEM((BPW, D), jnp.float32),
        pltpu.VMEM((BPW,), jnp.float32),
        pltpu.SemaphoreType.DMA,
    ],
)


def kernel(e_idc, p_idc, u_idc, feature_embeddings, node_embeddings,
           edge_embeddings):
    del feature_embeddings
    return _distmult(e_idc, p_idc, u_idc, node_embeddings, edge_embeddings)


# speedup vs baseline: 2.4338x; 2.4338x over previous
import jax
import jax.numpy as jnp
from jax import lax
from jax.experimental import pallas as pl
from jax.experimental.pallas import tpu as pltpu
from jax.experimental.pallas import tpu_sc as plsc

B = 16384
D = 64
NUM_ENTITIES = 1000000
NUM_RELATIONS = 1000

_info = plsc.get_sparse_core_info()
NC, NS, L = _info.num_cores, _info.num_subcores, _info.num_lanes
NW = NC * NS
BPW = B // NW
C = 16
NCH = BPW // C
NPAIR = NCH // 2
NCOL = D // L

_GATHER_DNUMS = lax.GatherDimensionNumbers(
    offset_dims=(), collapsed_slice_dims=(0,), start_index_map=(0,))


def _shuffle(x, idx):
    return lax.gather(
        x, idx[:, None], _GATHER_DNUMS, slice_sizes=(1,),
        mode=lax.GatherScatterMode.PROMISE_IN_BOUNDS)


def _distmult_body(e_idc, p_idc, u_idc, node3, edge_flat, out_hbm,
                   eidx, pidx, uidx, e0, e1, u0, u1, edge_v, out_v, s0, s1):
    wid = lax.axis_index("s") * NC + lax.axis_index("c")
    base = wid * BPW

    pltpu.sync_copy(e_idc.at[pl.ds(base, BPW)], eidx)
    pltpu.sync_copy(p_idc.at[pl.ds(base, BPW)], pidx)
    pltpu.sync_copy(u_idc.at[pl.ds(base, BPW)], uidx)
    pltpu.sync_copy(edge_flat, edge_v)

    def start(ch, ebuf, ubuf, sem):
        s = pl.ds(ch * C, C)
        ev = eidx[s]
        uv = uidx[s]
        for r in range(C):
            e_r = ev[r]
            pltpu.async_copy(node3.at[e_r >> 3, e_r & 7], ebuf.at[r], sem)
            u_r = uv[r]
            pltpu.async_copy(node3.at[u_r >> 3, u_r & 7], ubuf.at[r], sem)

    def drain(ebuf, ubuf, sem):
        dummy = node3.at[0, 0]
        for r in range(C):
            pltpu.make_async_copy(dummy, ebuf.at[r], sem).wait()
            pltpu.make_async_copy(dummy, ubuf.at[r], sem).wait()

    lane = lax.iota(jnp.int32, L)

    def compute(ch, ebuf, ubuf):
        s = pl.ds(ch * C, C)
        pv = pidx[s] * D
        tot = jnp.zeros((L,), jnp.float32)
        for r in range(C):
            pb_r = pv[r]
            acc = None
            for c in range(NCOL):
                d = pl.ds(c * L, L)
                t = ebuf[r, d] * edge_v[pl.ds(pb_r + c * L, L)] * ubuf[r, d]
                acc = t if acc is None else acc + t
            for sh in (8, 4, 2, 1):
                acc = acc + _shuffle(acc, lane ^ sh)
            tot = jnp.where(lane == r, acc, tot)
        out_v[s] = tot

    start(0, e0, u0, s0)
    start(1, e1, u1, s1)

    def pair(k, carry):
        ch0 = 2 * k
        drain(e0, u0, s0)
        compute(ch0, e0, u0)

        @pl.when(k < NPAIR - 1)
        def _():
            start(ch0 + 2, e0, u0, s0)

        drain(e1, u1, s1)
        compute(ch0 + 1, e1, u1)

        @pl.when(k < NPAIR - 1)
        def _():
            start(ch0 + 3, e1, u1, s1)

        return carry

    lax.fori_loop(0, NPAIR, pair, 0)

    pltpu.sync_copy(out_v, out_hbm.at[pl.ds(base, BPW)])


_distmult = pl.kernel(
    _distmult_body,
    out_type=jax.ShapeDtypeStruct((B,), jnp.float32),
    mesh=plsc.VectorSubcoreMesh(core_axis_name="c", subcore_axis_name="s"),
    scratch_types=[
        pltpu.VMEM((BPW,), jnp.int32),
        pltpu.VMEM((BPW,), jnp.int32),
        pltpu.VMEM((BPW,), jnp.int32),
        pltpu.VMEM((C, D), jnp.float32),
        pltpu.VMEM((C, D), jnp.float32),
        pltpu.VMEM((C, D), jnp.float32),
        pltpu.VMEM((C, D), jnp.float32),
        pltpu.VMEM((NUM_RELATIONS * D,), jnp.float32),
        pltpu.VMEM((BPW,), jnp.float32),
        pltpu.SemaphoreType.DMA,
        pltpu.SemaphoreType.DMA,
    ],
)


def kernel(e_idc, p_idc, u_idc, feature_embeddings, node_embeddings,
           edge_embeddings):
    del feature_embeddings
    node3 = node_embeddings.reshape(NUM_ENTITIES // 8, 8, D)
    edge_flat = edge_embeddings.reshape(NUM_RELATIONS * D)
    return _distmult(e_idc, p_idc, u_idc, node3, edge_flat)
